# baseline (device time: 178686 ns/iter reference)
import jax
import jax.numpy as jnp
from jax import lax
from jax.experimental import pallas as pl
from jax.experimental.pallas import tpu as pltpu

N_DEV = 4
B = 4
SQ = 512
SKV = 2048
D = 1024
HQ_LOC = 8
DH = 128
VN = 2 * DH
SCALE = 0.08838834764831843


def kernel(x, Wq, Wo, K_ext, V_ext):
    xb16 = x[0].astype(jnp.bfloat16)
    wq16 = Wq.astype(jnp.bfloat16).reshape(D, HQ_LOC, DH).transpose(1, 0, 2)
    wo16 = Wo.astype(jnp.bfloat16).reshape(HQ_LOC, DH, D)

    def body(x_ref, wq_ref, wo_ref, k_hbm, v_hbm, out_ref,
             xg_ref, kbuf, vbuf, vcat, pacc_ref, psend_ref, prec_ref,
             ag_send_sems, ag_recv_sems, rs_send_sems, rs_recv_sems,
             ksems, vsems, dma_sems):
        me = lax.axis_index("i")
        border = [lax.rem(me + 1 + k, N_DEV) for k in range(N_DEV)]
        hoff = me * HQ_LOC

        def start_kv(b, h, slot):
            pltpu.make_async_copy(
                k_hbm.at[b, :, pl.ds(hoff + h, 1), :], kbuf.at[slot],
                ksems.at[slot]).start()
            pltpu.make_async_copy(
                v_hbm.at[b, :, pl.ds(hoff + h, 1), :], vbuf.at[slot],
                vsems.at[slot]).start()

        def wait_kv(slot):
            pltpu.make_async_copy(
                k_hbm.at[0, :, pl.ds(0, 1), :], kbuf.at[slot],
                ksems.at[slot]).wait()
            pltpu.make_async_copy(
                v_hbm.at[0, :, pl.ds(0, 1), :], vbuf.at[slot],
                vsems.at[slot]).wait()

        start_kv(border[0], 0, 0)
        start_kv(border[0], 1, 1)
        ones_col = jnp.where(
            lax.broadcasted_iota(jnp.int32, (SKV, VN - DH), 1) == 0,
            1.0, 0.0).astype(jnp.bfloat16)
        vcat[0, :, DH:] = ones_col
        vcat[1, :, DH:] = ones_col

        bsem = pltpu.get_barrier_semaphore()
        for p in range(1, N_DEV):
            peer = lax.rem(me + p, N_DEV)
            pl.semaphore_signal(bsem, inc=1, device_id=(peer,),
                                device_id_type=pl.DeviceIdType.MESH)
        pl.semaphore_wait(bsem, N_DEV - 1)

        for p in range(1, N_DEV):
            peer = lax.rem(me + p, N_DEV)
            pltpu.make_async_remote_copy(
                src_ref=x_ref,
                dst_ref=xg_ref.at[me],
                send_sem=ag_send_sems.at[p - 1],
                recv_sem=ag_recv_sems.at[me],
                device_id=(peer,),
                device_id_type=pl.DeviceIdType.MESH,
            ).start()
        own_cp = pltpu.make_async_copy(x_ref, xg_ref.at[me], dma_sems.at[0])
        own_cp.start()

        for k in range(N_DEV):
            b_t = border[k]
            if k < N_DEV - 1:
                pltpu.make_async_remote_copy(
                    src_ref=x_ref, dst_ref=xg_ref.at[b_t],
                    send_sem=ag_send_sems.at[0],
                    recv_sem=ag_recv_sems.at[b_t],
                    device_id=(0,), device_id_type=pl.DeviceIdType.MESH,
                ).wait_recv()
            else:
                own_cp.wait()

            pacc_ref[...] = jnp.zeros((SQ, D), jnp.float32)
            b_next = border[k + 1] if k + 1 < N_DEV else None

            def pair_body(i, carry, b_t=b_t, b_next=b_next):
                for slot in range(2):
                    h = 2 * i + slot
                    wait_kv(slot)
                    kh = kbuf[slot, :, 0, :].astype(jnp.bfloat16)
                    vcat[slot, :, :DH] = vbuf[slot, :, 0, :].astype(
                        jnp.bfloat16)
                    nh = h + 2
                    @pl.when(nh < HQ_LOC)
                    def _():
                        start_kv(b_t, nh, slot)
                    if b_next is not None:
                        @pl.when(nh >= HQ_LOC)
                        def _():
                            start_kv(b_next, nh - HQ_LOC, slot)

                    qh = jnp.dot(xg_ref[b_t], wq_ref[h],
                                 preferred_element_type=jnp.float32)
                    qh = (qh * SCALE).astype(jnp.bfloat16)
                    s = lax.dot_general(
                        qh, kh, (((1,), (1,)), ((), ())),
                        preferred_element_type=jnp.float32)
                    e16 = jnp.exp(s).astype(jnp.bfloat16)
                    ovl = jnp.dot(e16, vcat[slot],
                                  preferred_element_type=jnp.float32)
                    ov = ovl[:, :DH]
                    l = ovl[:, DH:DH + 1]
                    oh = (ov / l).astype(jnp.bfloat16)
                    pacc_ref[...] += jnp.dot(
                        oh, wo_ref[h], preferred_element_type=jnp.float32)
                return carry

            lax.fori_loop(0, HQ_LOC // 2, pair_body, 0)

            if k < N_DEV - 1:
                psend_ref[k] = pacc_ref[...].astype(jnp.bfloat16)
                pltpu.make_async_remote_copy(
                    src_ref=psend_ref.at[k],
                    dst_ref=prec_ref.at[me],
                    send_sem=rs_send_sems.at[k],
                    recv_sem=rs_recv_sems.at[me],
                    device_id=(b_t,),
                    device_id_type=pl.DeviceIdType.MESH,
                ).start()

        for k in range(N_DEV - 1):
            src = border[k]
            pltpu.make_async_remote_copy(
                src_ref=psend_ref.at[0], dst_ref=prec_ref.at[src],
                send_sem=rs_send_sems.at[0],
                recv_sem=rs_recv_sems.at[src],
                device_id=(0,), device_id_type=pl.DeviceIdType.MESH,
            ).wait_recv()

        out_ref[0] = (
            (pacc_ref[...] + prec_ref[border[0]].astype(jnp.float32))
            + (prec_ref[border[1]].astype(jnp.float32)
               + prec_ref[border[2]].astype(jnp.float32))
        )

        for p in range(1, N_DEV):
            pltpu.make_async_remote_copy(
                src_ref=x_ref, dst_ref=xg_ref.at[0],
                send_sem=ag_send_sems.at[p - 1], recv_sem=ag_recv_sems.at[0],
                device_id=(0,), device_id_type=pl.DeviceIdType.MESH,
            ).wait_send()
        for k in range(N_DEV - 1):
            pltpu.make_async_remote_copy(
                src_ref=psend_ref.at[k], dst_ref=prec_ref.at[0],
                send_sem=rs_send_sems.at[k], recv_sem=rs_recv_sems.at[0],
                device_id=(0,), device_id_type=pl.DeviceIdType.MESH,
            ).wait_send()

    return pl.pallas_call(
        body,
        out_shape=jax.ShapeDtypeStruct((1, SQ, D), jnp.float32),
        in_specs=[
            pl.BlockSpec(memory_space=pltpu.MemorySpace.VMEM),
            pl.BlockSpec(memory_space=pltpu.MemorySpace.VMEM),
            pl.BlockSpec(memory_space=pltpu.MemorySpace.VMEM),
            pl.BlockSpec(memory_space=pltpu.MemorySpace.HBM),
            pl.BlockSpec(memory_space=pltpu.MemorySpace.HBM),
        ],
        out_specs=pl.BlockSpec(memory_space=pltpu.MemorySpace.VMEM),
        scratch_shapes=[
            pltpu.VMEM((N_DEV, SQ, D), jnp.bfloat16),
            pltpu.VMEM((2, SKV, 1, DH), jnp.float32),
            pltpu.VMEM((2, SKV, 1, DH), jnp.float32),
            pltpu.VMEM((2, SKV, VN), jnp.bfloat16),
            pltpu.VMEM((SQ, D), jnp.float32),
            pltpu.VMEM((N_DEV - 1, SQ, D), jnp.bfloat16),
            pltpu.VMEM((N_DEV, SQ, D), jnp.bfloat16),
            pltpu.SemaphoreType.DMA((N_DEV,)),
            pltpu.SemaphoreType.DMA((N_DEV,)),
            pltpu.SemaphoreType.DMA((N_DEV,)),
            pltpu.SemaphoreType.DMA((N_DEV,)),
            pltpu.SemaphoreType.DMA((2,)),
            pltpu.SemaphoreType.DMA((2,)),
            pltpu.SemaphoreType.DMA((2,)),
        ],
        compiler_params=pltpu.CompilerParams(
            collective_id=0, vmem_limit_bytes=60 * 1024 * 1024),
    )(xb16, wq16, wo16, K_ext, V_ext)
